# baseline (speedup 1.0000x reference)
import jax
import jax.numpy as jnp
from jax.experimental import pallas as pl
from jax.experimental.pallas import tpu as pltpu

BATCH = 1024
DIM_IN = 256
DIM_OUT = 256
NUM_X = 64
IB = 64
NSTEPS = DIM_IN // IB


def _interp_matmul_kernel(params_ref, x_ref, y_ref, out_ref):
    s = pl.program_id(0)
    xmin = params_ref[0, 0]
    inv_h = params_ref[0, 1]
    u = (x_ref[0] - xmin) * inv_h
    uc = jnp.clip(u, 0.0, float(NUM_X - 1))

    yb = y_ref[...].reshape(IB, DIM_OUT, NUM_X).astype(jnp.bfloat16)
    yt = jnp.swapaxes(jnp.transpose(yb, (0, 2, 1)), 0, 1)
    arow = jnp.sum(yt.astype(jnp.float32), axis=(0, 1))

    u2 = jnp.concatenate([uc, uc], axis=1)
    half = (jax.lax.broadcasted_iota(jnp.int32, (1, 2 * IB), 1) >= IB).astype(
        jnp.float32
    )
    mslices = []
    for k in range(0, NUM_X, 2):
        d = (u2 - (half + float(k))).astype(jnp.bfloat16)
        mslices.append(jnp.clip(jnp.abs(d), jnp.bfloat16(0.0), jnp.bfloat16(1.0)))
    mm = jnp.concatenate(mslices, axis=1)

    acc = jax.lax.dot_general(
        mm,
        yt.reshape(NUM_X * IB, DIM_OUT),
        (((1,), (0,)), ((), ())),
        preferred_element_type=jnp.float32,
    )

    e0 = jnp.minimum(u, 0.0).astype(jnp.bfloat16)
    e1 = jnp.maximum(u - float(NUM_X - 1), 0.0).astype(jnp.bfloat16)
    d_lo = yt[1] - yt[0]
    d_hi = yt[NUM_X - 1] - yt[NUM_X - 2]
    corr = jax.lax.dot_general(
        e0, d_lo, (((1,), (0,)), ((), ())),
        preferred_element_type=jnp.float32,
    ) + jax.lax.dot_general(
        e1, d_hi, (((1,), (0,)), ((), ())),
        preferred_element_type=jnp.float32,
    )

    step_out = (arow[None, :] + corr) - acc

    @pl.when(s == 0)
    def _first():
        out_ref[...] = step_out

    @pl.when(s > 0)
    def _rest():
        out_ref[...] += step_out


@jax.jit
def kernel(x, X, Y):
    xmin = X[0]
    inv_h = (NUM_X - 1) / (X[NUM_X - 1] - X[0])
    params = jnp.stack([xmin, inv_h]).reshape(1, 2)
    xb = jnp.transpose(x.reshape(BATCH, NSTEPS, IB), (1, 0, 2))

    out = pl.pallas_call(
        _interp_matmul_kernel,
        grid=(NSTEPS,),
        in_specs=[
            pl.BlockSpec(memory_space=pltpu.SMEM),
            pl.BlockSpec((1, BATCH, IB), lambda s: (s, 0, 0)),
            pl.BlockSpec((IB, DIM_OUT * NUM_X), lambda s: (s, 0)),
        ],
        out_specs=pl.BlockSpec((BATCH, DIM_OUT), lambda s: (0, 0)),
        out_shape=jax.ShapeDtypeStruct((BATCH, DIM_OUT), jnp.float32),
    )(params, xb, Y.reshape(DIM_IN, DIM_OUT * NUM_X))
    return out


# speedup vs baseline: 2.1339x; 2.1339x over previous
import jax
import jax.numpy as jnp
from jax.experimental import pallas as pl
from jax.experimental.pallas import tpu as pltpu

BATCH = 1024
DIM_IN = 256
DIM_OUT = 256
NUM_X = 64
KB = 32
NSTEPS = NUM_X // KB


def _interp_matmul_kernel(params_ref, x_ref, yn_ref, out_ref):
    s = pl.program_id(0)
    xmin = params_ref[0, 0]
    inv_h = params_ref[0, 1]
    u = (x_ref[...] - xmin) * inv_h
    uc = jnp.clip(u, 0.0, float(NUM_X - 1))
    base = (s * KB).astype(jnp.float32)

    mslices = []
    for j in range(0, KB, 2):
        d0 = (uc - (base + float(j))).astype(jnp.bfloat16)
        d1 = d0 - jnp.bfloat16(1.0)
        mslices.append(jnp.clip(jnp.abs(d0), jnp.bfloat16(0.0), jnp.bfloat16(1.0)))
        mslices.append(jnp.clip(jnp.abs(d1), jnp.bfloat16(0.0), jnp.bfloat16(1.0)))
    mm = jnp.concatenate(mslices, axis=1)

    acc = jax.lax.dot_general(
        mm,
        yn_ref[...].reshape(KB * DIM_IN, DIM_OUT),
        (((1,), (0,)), ((), ())),
        preferred_element_type=jnp.float32,
    )
    arow = -jnp.sum(yn_ref[...].astype(jnp.float32), axis=(0, 1))
    step_out = acc + arow[None, :]

    @pl.when(s == 0)
    def _first():
        e0 = jnp.minimum(u, 0.0).astype(jnp.bfloat16)
        d0 = yn_ref[0] - yn_ref[1]
        corr = jax.lax.dot_general(
            e0, d0, (((1,), (0,)), ((), ())),
            preferred_element_type=jnp.float32,
        )
        out_ref[...] = step_out + corr

    @pl.when(jnp.logical_and(s > 0, s < NSTEPS - 1))
    def _mid():
        out_ref[...] += step_out

    @pl.when(s == NSTEPS - 1)
    def _last():
        e1 = jnp.maximum(u - float(NUM_X - 1), 0.0).astype(jnp.bfloat16)
        d1 = yn_ref[KB - 2] - yn_ref[KB - 1]
        corr = jax.lax.dot_general(
            e1, d1, (((1,), (0,)), ((), ())),
            preferred_element_type=jnp.float32,
        )
        out_ref[...] += step_out + corr


@jax.jit
def kernel(x, X, Y):
    xmin = X[0]
    inv_h = (NUM_X - 1) / (X[NUM_X - 1] - X[0])
    params = jnp.stack([xmin, inv_h]).reshape(1, 2)
    yneg = (-jnp.transpose(Y, (2, 0, 1))).astype(jnp.bfloat16)

    out = pl.pallas_call(
        _interp_matmul_kernel,
        grid=(NSTEPS,),
        in_specs=[
            pl.BlockSpec(memory_space=pltpu.SMEM),
            pl.BlockSpec((BATCH, DIM_IN), lambda s: (0, 0)),
            pl.BlockSpec((KB, DIM_IN, DIM_OUT), lambda s: (s, 0, 0)),
        ],
        out_specs=pl.BlockSpec((BATCH, DIM_OUT), lambda s: (0, 0)),
        out_shape=jax.ShapeDtypeStruct((BATCH, DIM_OUT), jnp.float32),
    )(params, x, yneg)
    return out


# speedup vs baseline: 2.1487x; 1.0069x over previous
import jax
import jax.numpy as jnp
from jax.experimental import pallas as pl
from jax.experimental.pallas import tpu as pltpu

BATCH = 1024
DIM_IN = 256
DIM_OUT = 256
NUM_X = 64
KB = 16
NSTEPS = NUM_X // KB


def _interp_matmul_kernel(params_ref, x_ref, yn_ref, out_ref):
    s = pl.program_id(0)
    xmin = params_ref[0, 0]
    inv_h = params_ref[0, 1]
    u = (x_ref[...] - xmin) * inv_h
    uc = jnp.clip(u, 0.0, float(NUM_X - 1))
    base = (s * KB).astype(jnp.float32)

    mslices = []
    for j in range(0, KB, 2):
        d0 = (uc - (base + float(j))).astype(jnp.bfloat16)
        d1 = d0 - jnp.bfloat16(1.0)
        mslices.append(jnp.clip(jnp.abs(d0), jnp.bfloat16(0.0), jnp.bfloat16(1.0)))
        mslices.append(jnp.clip(jnp.abs(d1), jnp.bfloat16(0.0), jnp.bfloat16(1.0)))
    mm = jnp.concatenate(mslices, axis=1)

    acc = jax.lax.dot_general(
        mm,
        yn_ref[...].reshape(KB * DIM_IN, DIM_OUT),
        (((1,), (0,)), ((), ())),
        preferred_element_type=jnp.float32,
    )
    arow = -jnp.sum(yn_ref[...].astype(jnp.float32), axis=(0, 1))
    step_out = acc + arow[None, :]

    @pl.when(s == 0)
    def _first():
        e0 = jnp.minimum(u, 0.0).astype(jnp.bfloat16)
        d0 = yn_ref[0] - yn_ref[1]
        corr = jax.lax.dot_general(
            e0, d0, (((1,), (0,)), ((), ())),
            preferred_element_type=jnp.float32,
        )
        out_ref[...] = step_out + corr

    @pl.when(jnp.logical_and(s > 0, s < NSTEPS - 1))
    def _mid():
        out_ref[...] += step_out

    @pl.when(s == NSTEPS - 1)
    def _last():
        e1 = jnp.maximum(u - float(NUM_X - 1), 0.0).astype(jnp.bfloat16)
        d1 = yn_ref[KB - 2] - yn_ref[KB - 1]
        corr = jax.lax.dot_general(
            e1, d1, (((1,), (0,)), ((), ())),
            preferred_element_type=jnp.float32,
        )
        out_ref[...] += step_out + corr


@jax.jit
def kernel(x, X, Y):
    xmin = X[0]
    inv_h = (NUM_X - 1) / (X[NUM_X - 1] - X[0])
    params = jnp.stack([xmin, inv_h]).reshape(1, 2)
    yneg = (-jnp.transpose(Y, (2, 0, 1))).astype(jnp.bfloat16)

    out = pl.pallas_call(
        _interp_matmul_kernel,
        grid=(NSTEPS,),
        in_specs=[
            pl.BlockSpec(memory_space=pltpu.SMEM),
            pl.BlockSpec((BATCH, DIM_IN), lambda s: (0, 0)),
            pl.BlockSpec((KB, DIM_IN, DIM_OUT), lambda s: (s, 0, 0)),
        ],
        out_specs=pl.BlockSpec((BATCH, DIM_OUT), lambda s: (0, 0)),
        out_shape=jax.ShapeDtypeStruct((BATCH, DIM_OUT), jnp.float32),
    )(params, x, yneg)
    return out
